# baseline (speedup 1.0000x reference)
import jax
import jax.numpy as jnp
from jax import lax
from jax.experimental import pallas as pl
from jax.experimental.pallas import tpu as pltpu
from jax.experimental.pallas import tpu_sc as plsc

N_NODES_ = 10000
D_ = 128
DH = 64
DW = 72
K_ = 32
N_EDGES_ = 320000

CHUNK = 128
NBUF = 4
NCH0 = 160
NCH1 = 0
NCH_TOT = 16 * NCH0
EDGES_PAD = NCH_TOT * CHUNK
ROWS = 10240
DUMP_ROW = N_NODES_
ROWS_PER_TILE = ROWS // 16
WB_CHUNK = 128
BR = 400


def _matmul_maxk_body(feat_ref, ws_ref, wn_ref, hself_ref, h0_ref, h1_ref):
    f = feat_ref[...]
    hs = jnp.dot(f, ws_ref[...], preferred_element_type=jnp.float32)
    hn = jnp.dot(f, wn_ref[...], preferred_element_type=jnp.float32)
    hself_ref[...] = hs

    work = hn
    keep = jnp.zeros(hn.shape, dtype=jnp.bool_)
    for _ in range(K_):
        m = jnp.max(work, axis=1, keepdims=True)
        sel = work == m
        keep = jnp.logical_or(keep, sel)
        work = jnp.where(sel, -jnp.inf, work)
    hsp = jnp.where(keep, hn, 0.0)

    pad_cols = lax.broadcasted_iota(jnp.int32, (hn.shape[0], DW - DH), 1)
    ones_zeros = jnp.where(pad_cols == 0, 1.0, 0.0)
    h0_ref[...] = jnp.concatenate([hsp[:, :DH], ones_zeros], axis=1)
    h1_ref[...] = jnp.concatenate([hsp[:, DH:], ones_zeros], axis=1)


def _matmul_maxk(feat, w_self, w_neigh):
    n = feat.shape[0]
    return pl.pallas_call(
        _matmul_maxk_body,
        grid=(n // BR,),
        in_specs=[
            pl.BlockSpec((BR, D_), lambda i: (i, 0)),
            pl.BlockSpec((D_, D_), lambda i: (0, 0)),
            pl.BlockSpec((D_, D_), lambda i: (0, 0)),
        ],
        out_specs=[
            pl.BlockSpec((BR, D_), lambda i: (i, 0)),
            pl.BlockSpec((BR, DW), lambda i: (i, 0)),
            pl.BlockSpec((BR, DW), lambda i: (i, 0)),
        ],
        out_shape=[
            jax.ShapeDtypeStruct((n, D_), jnp.float32),
            jax.ShapeDtypeStruct((n, DW), jnp.float32),
            jax.ShapeDtypeStruct((n, DW), jnp.float32),
        ],
        compiler_params=pltpu.CompilerParams(
            dimension_semantics=("arbitrary",),
        ),
    )(feat, w_self, w_neigh)


def _sc_aggregate_body(h0_hbm, h1_hbm, src_hbm, dst_hbm, agg_hbm,
                       src_v, dst_v, rows_v, acc_s,
                       sem0, sem1, sem2, sem3):
    sems = [sem0, sem1, sem2, sem3]
    c = lax.axis_index("c")
    s = lax.axis_index("s")
    base = s * ROWS_PER_TILE

    start = jnp.where(c == 0, s * NCH0, 0)
    ngrp = jnp.where(c == 0, NCH0 // NBUF, NCH1 // NBUF)
    nch = ngrp * NBUF

    pltpu.sync_copy(src_hbm.at[pl.ds(start, NCH0)], src_v)
    pltpu.sync_copy(dst_hbm.at[pl.ds(start, NCH0)], dst_v)

    for p in range(2):
        h_hbm = h0_hbm if p == 0 else h1_hbm

        def _fill_zero(i, _):
            for j in range(DH // 16):
                rows_v[0, i, pl.ds(j * 16, 16)] = jnp.zeros((16,), jnp.float32)
            rows_v[0, i, pl.ds(DW - 16, 16)] = jnp.zeros((16,), jnp.float32)
            return 0

        lax.fori_loop(0, CHUNK, _fill_zero, 0)

        def _zero_acc(i, _):
            pltpu.sync_copy(rows_v.at[0],
                            acc_s.at[pl.ds(base + i * WB_CHUNK, WB_CHUNK)])
            return 0

        lax.fori_loop(0, ROWS_PER_TILE // WB_CHUNK, _zero_acc, 0)

        for b in range(NBUF):
            @pl.when(b < nch)
            def _():
                pltpu.async_copy(h_hbm.at[src_v.at[b]], rows_v.at[b], sems[b])

        plsc.subcore_barrier()

        def _edge_group(jj, _):
            for b in range(NBUF):
                j = jj * NBUF + b
                pltpu.make_async_copy(
                    h_hbm.at[src_v.at[j]], rows_v.at[b], sems[b]
                ).wait()
                pltpu.sync_copy(rows_v.at[b], acc_s.at[dst_v.at[j]], add=True)
                nj = j + NBUF

                @pl.when(nj < nch)
                def _():
                    pltpu.async_copy(
                        h_hbm.at[src_v.at[nj]], rows_v.at[b], sems[b]
                    )

            return 0

        lax.fori_loop(0, ngrp, _edge_group, 0)

        plsc.subcore_barrier()

        def _writeback(i, _):
            r = base + i * WB_CHUNK
            pltpu.sync_copy(acc_s.at[pl.ds(r, WB_CHUNK)], rows_v.at[0])
            pltpu.sync_copy(rows_v.at[0],
                            agg_hbm.at[c].at[p].at[pl.ds(r, WB_CHUNK)])
            return 0

        lax.fori_loop(0, ROWS_PER_TILE // WB_CHUNK, _writeback, 0)


def _sc_aggregate(h0, h1, src2d, dst2d):
    mesh = plsc.VectorSubcoreMesh(core_axis_name="c", subcore_axis_name="s")
    kern = pl.kernel(
        _sc_aggregate_body,
        mesh=mesh,
        out_type=jax.ShapeDtypeStruct((2, 2, ROWS, DW), jnp.float32),
        scratch_types=[
            pltpu.VMEM((NCH0, CHUNK), jnp.int32),
            pltpu.VMEM((NCH0, CHUNK), jnp.int32),
            pltpu.VMEM((NBUF, CHUNK, DW), jnp.float32),
            pltpu.VMEM_SHARED((ROWS, DW), jnp.float32),
            pltpu.SemaphoreType.DMA,
            pltpu.SemaphoreType.DMA,
            pltpu.SemaphoreType.DMA,
            pltpu.SemaphoreType.DMA,
        ],
        compiler_params=pltpu.CompilerParams(
            needs_layout_passes=False,
            use_tc_tiling_on_sc=False,
        ),
    )
    return kern(h0, h1, src2d, dst2d)


def _combine_body(hself_ref, agg_ref, out_ref):
    a_lo = agg_ref[0, 0, :, :DH] + agg_ref[1, 0, :, :DH]
    a_hi = agg_ref[0, 1, :, :DH] + agg_ref[1, 1, :, :DH]
    a = jnp.concatenate([a_lo, a_hi], axis=1)
    d = agg_ref[0, 0, :, DH:DH + 1] + agg_ref[1, 0, :, DH:DH + 1]
    out_ref[...] = hself_ref[...] + a / jnp.maximum(d, 1.0)


def _combine(h_self, agg):
    n = h_self.shape[0]
    return pl.pallas_call(
        _combine_body,
        grid=(n // BR,),
        in_specs=[
            pl.BlockSpec((BR, D_), lambda i: (i, 0)),
            pl.BlockSpec((2, 2, BR, DW), lambda i: (0, 0, i, 0)),
        ],
        out_specs=pl.BlockSpec((BR, D_), lambda i: (i, 0)),
        out_shape=jax.ShapeDtypeStruct((n, D_), jnp.float32),
        compiler_params=pltpu.CompilerParams(
            dimension_semantics=("arbitrary",),
        ),
    )(h_self, agg)


def kernel(feat, edge_index, W_self, W_neigh):
    h_self, h0, h1 = _matmul_maxk(feat, W_self, W_neigh)

    src = edge_index[0]
    dst = edge_index[1]
    pad = EDGES_PAD - N_EDGES_
    src_p = jnp.concatenate([src, jnp.zeros((pad,), jnp.int32)])
    dump = DUMP_ROW + (jnp.arange(pad, dtype=jnp.int32) % (ROWS - N_NODES_))
    dst_p = jnp.concatenate([dst, dump])
    src2d = src_p.reshape(NCH_TOT, CHUNK)
    dst2d = dst_p.reshape(NCH_TOT, CHUNK)

    agg = _sc_aggregate(h0, h1, src2d, dst2d)
    return _combine(h_self, agg)


# speedup vs baseline: 1.2656x; 1.2656x over previous
import jax
import jax.numpy as jnp
from jax import lax
from jax.experimental import pallas as pl
from jax.experimental.pallas import tpu as pltpu
from jax.experimental.pallas import tpu_sc as plsc

N_NODES_ = 10000
D_ = 128
DH = 64
DW = 72
K_ = 32
N_EDGES_ = 320000

CHUNK = 128
NBUF = 4
NCH0 = 144
NCH1 = 16
NCH_REAL = 16 * (NCH0 + NCH1)
NCH_TOT = 2688
EDGES_PAD = NCH_TOT * CHUNK
ROWS = 10240
DUMP_ROW = N_NODES_
ROWS_PER_TILE = ROWS // 16
WB_CHUNK = 128
BR = 512


def _matmul_maxk_body(feat_ref, ws_ref, wn_ref, hself_ref, h0_ref, h1_ref):
    f = feat_ref[...]
    hs = jnp.dot(f, ws_ref[...], preferred_element_type=jnp.float32)
    hn = jnp.dot(f, wn_ref[...], preferred_element_type=jnp.float32)
    hself_ref[...] = hs

    work = hn
    keep = jnp.zeros(hn.shape, dtype=jnp.bool_)
    for _ in range(K_):
        m = jnp.max(work, axis=1, keepdims=True)
        sel = work == m
        keep = jnp.logical_or(keep, sel)
        work = jnp.where(sel, -jnp.inf, work)
    hsp = jnp.where(keep, hn, 0.0)

    pad_cols = lax.broadcasted_iota(jnp.int32, (hn.shape[0], DW - DH), 1)
    ones_zeros = jnp.where(pad_cols == 0, 1.0, 0.0)
    h0_ref[...] = jnp.concatenate([hsp[:, :DH], ones_zeros], axis=1)
    h1_ref[...] = jnp.concatenate([hsp[:, DH:], ones_zeros], axis=1)


def _matmul_maxk(feat_p, w_self, w_neigh):
    return pl.pallas_call(
        _matmul_maxk_body,
        grid=(ROWS // BR,),
        in_specs=[
            pl.BlockSpec((BR, D_), lambda i: (i, 0)),
            pl.BlockSpec((D_, D_), lambda i: (0, 0)),
            pl.BlockSpec((D_, D_), lambda i: (0, 0)),
        ],
        out_specs=[
            pl.BlockSpec((BR, D_), lambda i: (i, 0)),
            pl.BlockSpec((BR, DW), lambda i: (i, 0)),
            pl.BlockSpec((BR, DW), lambda i: (i, 0)),
        ],
        out_shape=[
            jax.ShapeDtypeStruct((ROWS, D_), jnp.float32),
            jax.ShapeDtypeStruct((ROWS, DW), jnp.float32),
            jax.ShapeDtypeStruct((ROWS, DW), jnp.float32),
        ],
        compiler_params=pltpu.CompilerParams(
            dimension_semantics=("arbitrary",),
        ),
    )(feat_p, w_self, w_neigh)


def _sc_aggregate_body(h0_hbm, h1_hbm, src_hbm, dst_hbm, agg_hbm,
                       src_v, dst_v, rows_v, acc_s,
                       sem0, sem1, sem2, sem3):
    sems = [sem0, sem1, sem2, sem3]
    c = lax.axis_index("c")
    s = lax.axis_index("s")
    base = s * ROWS_PER_TILE

    start = jnp.where(c == 0, s * NCH0, 16 * NCH0 + s * NCH1)
    ngrp = jnp.where(c == 0, NCH0 // NBUF, NCH1 // NBUF)
    nch = ngrp * NBUF

    pltpu.sync_copy(src_hbm.at[pl.ds(start, NCH0)], src_v)
    pltpu.sync_copy(dst_hbm.at[pl.ds(start, NCH0)], dst_v)

    for p in range(2):
        h_hbm = h0_hbm if p == 0 else h1_hbm

        def _fill_zero(i, _):
            for j in range(DH // 16):
                rows_v[0, i, pl.ds(j * 16, 16)] = jnp.zeros((16,), jnp.float32)
            rows_v[0, i, pl.ds(DW - 16, 16)] = jnp.zeros((16,), jnp.float32)
            return 0

        lax.fori_loop(0, CHUNK, _fill_zero, 0)

        def _zero_acc(i, _):
            pltpu.sync_copy(rows_v.at[0],
                            acc_s.at[pl.ds(base + i * WB_CHUNK, WB_CHUNK)])
            return 0

        lax.fori_loop(0, ROWS_PER_TILE // WB_CHUNK, _zero_acc, 0)

        for b in range(NBUF):
            pltpu.async_copy(h_hbm.at[src_v.at[b]], rows_v.at[b], sems[b])

        plsc.subcore_barrier()

        def _edge_group(jj, _):
            for b in range(NBUF):
                j = jj * NBUF + b
                pltpu.make_async_copy(
                    h_hbm.at[src_v.at[j]], rows_v.at[b], sems[b]
                ).wait()
                pltpu.sync_copy(rows_v.at[b], acc_s.at[dst_v.at[j]], add=True)
                nj = j + NBUF

                @pl.when(nj < nch)
                def _():
                    pltpu.async_copy(
                        h_hbm.at[src_v.at[nj]], rows_v.at[b], sems[b]
                    )

            return 0

        lax.fori_loop(0, ngrp, _edge_group, 0)

        plsc.subcore_barrier()

        def _writeback(i, _):
            r = base + i * WB_CHUNK
            pltpu.sync_copy(acc_s.at[pl.ds(r, WB_CHUNK)], rows_v.at[0])
            pltpu.sync_copy(rows_v.at[0],
                            agg_hbm.at[c].at[p].at[pl.ds(r, WB_CHUNK)])
            return 0

        lax.fori_loop(0, ROWS_PER_TILE // WB_CHUNK, _writeback, 0)


def _sc_aggregate(h0, h1, src2d, dst2d):
    mesh = plsc.VectorSubcoreMesh(core_axis_name="c", subcore_axis_name="s")
    kern = pl.kernel(
        _sc_aggregate_body,
        mesh=mesh,
        out_type=jax.ShapeDtypeStruct((2, 2, ROWS, DW), jnp.float32),
        scratch_types=[
            pltpu.VMEM((NCH0, CHUNK), jnp.int32),
            pltpu.VMEM((NCH0, CHUNK), jnp.int32),
            pltpu.VMEM((NBUF, CHUNK, DW), jnp.float32),
            pltpu.VMEM_SHARED((ROWS, DW), jnp.float32),
            pltpu.SemaphoreType.DMA,
            pltpu.SemaphoreType.DMA,
            pltpu.SemaphoreType.DMA,
            pltpu.SemaphoreType.DMA,
        ],
        compiler_params=pltpu.CompilerParams(
            needs_layout_passes=False,
            use_tc_tiling_on_sc=False,
        ),
    )
    return kern(h0, h1, src2d, dst2d)


def _combine_body(hself_ref, agg_ref, out_ref):
    a_lo = agg_ref[0, 0, :, :DH] + agg_ref[1, 0, :, :DH]
    a_hi = agg_ref[0, 1, :, :DH] + agg_ref[1, 1, :, :DH]
    a = jnp.concatenate([a_lo, a_hi], axis=1)
    d = agg_ref[0, 0, :, DH:DH + 1] + agg_ref[1, 0, :, DH:DH + 1]
    out_ref[...] = hself_ref[...] + a / jnp.maximum(d, 1.0)


def _combine(h_self, agg):
    return pl.pallas_call(
        _combine_body,
        grid=(ROWS // BR,),
        in_specs=[
            pl.BlockSpec((BR, D_), lambda i: (i, 0)),
            pl.BlockSpec((2, 2, BR, DW), lambda i: (0, 0, i, 0)),
        ],
        out_specs=pl.BlockSpec((BR, D_), lambda i: (i, 0)),
        out_shape=jax.ShapeDtypeStruct((ROWS, D_), jnp.float32),
        compiler_params=pltpu.CompilerParams(
            dimension_semantics=("arbitrary",),
        ),
    )(h_self, agg)


def kernel(feat, edge_index, W_self, W_neigh):
    feat_p = jnp.pad(feat, ((0, ROWS - N_NODES_), (0, 0)))
    h_self, h0, h1 = _matmul_maxk(feat_p, W_self, W_neigh)

    src = edge_index[0]
    dst = edge_index[1]
    pad = EDGES_PAD - N_EDGES_
    src_p = jnp.concatenate([src, jnp.zeros((pad,), jnp.int32)])
    dump = DUMP_ROW + (jnp.arange(pad, dtype=jnp.int32) % (ROWS - N_NODES_))
    dst_p = jnp.concatenate([dst, dump])
    src2d = src_p.reshape(NCH_TOT, CHUNK)
    dst2d = dst_p.reshape(NCH_TOT, CHUNK)

    agg = _sc_aggregate(h0, h1, src2d, dst2d)
    out = _combine(h_self, agg)
    return out[:N_NODES_]


# speedup vs baseline: 1.2788x; 1.0105x over previous
import jax
import jax.numpy as jnp
from jax import lax
from jax.experimental import pallas as pl
from jax.experimental.pallas import tpu as pltpu
from jax.experimental.pallas import tpu_sc as plsc

N_NODES_ = 10000
D_ = 128
DH = 64
DW = 72
K_ = 32
N_EDGES_ = 320000

CHUNK = 128
NBUF = 4
NCH0 = 152
NCH1 = 8
NCH_REAL = 16 * (NCH0 + NCH1)
NCH_TOT = 2688
EDGES_PAD = NCH_TOT * CHUNK
ROWS = 10240
DUMP_ROW = N_NODES_
ROWS_PER_TILE = ROWS // 16
WB_CHUNK = 128
BR = 512


def _matmul_maxk_body(feat_ref, ws_ref, wn_ref, hself_ref, h0_ref, h1_ref):
    f = feat_ref[...]
    hs = jnp.dot(f, ws_ref[...], preferred_element_type=jnp.float32)
    hn = jnp.dot(f, wn_ref[...], preferred_element_type=jnp.float32)
    hself_ref[...] = hs

    work = hn
    keep = jnp.zeros(hn.shape, dtype=jnp.bool_)
    for _ in range(K_):
        m = jnp.max(work, axis=1, keepdims=True)
        sel = work == m
        keep = jnp.logical_or(keep, sel)
        work = jnp.where(sel, -jnp.inf, work)
    hsp = jnp.where(keep, hn, 0.0)

    pad_cols = lax.broadcasted_iota(jnp.int32, (hn.shape[0], DW - DH), 1)
    ones_zeros = jnp.where(pad_cols == 0, 1.0, 0.0)
    h0_ref[...] = jnp.concatenate([hsp[:, :DH], ones_zeros], axis=1)
    h1_ref[...] = jnp.concatenate([hsp[:, DH:], ones_zeros], axis=1)


def _matmul_maxk(feat_p, w_self, w_neigh):
    return pl.pallas_call(
        _matmul_maxk_body,
        grid=(ROWS // BR,),
        in_specs=[
            pl.BlockSpec((BR, D_), lambda i: (i, 0)),
            pl.BlockSpec((D_, D_), lambda i: (0, 0)),
            pl.BlockSpec((D_, D_), lambda i: (0, 0)),
        ],
        out_specs=[
            pl.BlockSpec((BR, D_), lambda i: (i, 0)),
            pl.BlockSpec((BR, DW), lambda i: (i, 0)),
            pl.BlockSpec((BR, DW), lambda i: (i, 0)),
        ],
        out_shape=[
            jax.ShapeDtypeStruct((ROWS, D_), jnp.float32),
            jax.ShapeDtypeStruct((ROWS, DW), jnp.float32),
            jax.ShapeDtypeStruct((ROWS, DW), jnp.float32),
        ],
        compiler_params=pltpu.CompilerParams(
            dimension_semantics=("arbitrary",),
        ),
    )(feat_p, w_self, w_neigh)


def _sc_aggregate_body(h0_hbm, h1_hbm, src_hbm, dst_hbm, agg_hbm,
                       src_v, dst_v, rows_v, acc_s,
                       sem0, sem1, sem2, sem3):
    sems = [sem0, sem1, sem2, sem3]
    c = lax.axis_index("c")
    s = lax.axis_index("s")
    base = s * ROWS_PER_TILE

    start = jnp.where(c == 0, s * NCH0, 16 * NCH0 + s * NCH1)
    ngrp = jnp.where(c == 0, NCH0 // NBUF, NCH1 // NBUF)
    nch = ngrp * NBUF

    pltpu.sync_copy(src_hbm.at[pl.ds(start, NCH0)], src_v)
    pltpu.sync_copy(dst_hbm.at[pl.ds(start, NCH0)], dst_v)

    for p in range(2):
        h_hbm = h0_hbm if p == 0 else h1_hbm

        def _fill_zero(i, _):
            for j in range(DH // 16):
                rows_v[0, i, pl.ds(j * 16, 16)] = jnp.zeros((16,), jnp.float32)
            rows_v[0, i, pl.ds(DW - 16, 16)] = jnp.zeros((16,), jnp.float32)
            return 0

        lax.fori_loop(0, CHUNK, _fill_zero, 0)

        def _zero_acc(i, _):
            pltpu.sync_copy(rows_v.at[0],
                            acc_s.at[pl.ds(base + i * WB_CHUNK, WB_CHUNK)])
            return 0

        lax.fori_loop(0, ROWS_PER_TILE // WB_CHUNK, _zero_acc, 0)

        for b in range(NBUF):
            pltpu.async_copy(h_hbm.at[src_v.at[b]], rows_v.at[b], sems[b])

        plsc.subcore_barrier()

        def _edge_group(jj, _):
            for b in range(NBUF):
                j = jj * NBUF + b
                pltpu.make_async_copy(
                    h_hbm.at[src_v.at[j]], rows_v.at[b], sems[b]
                ).wait()
                pltpu.sync_copy(rows_v.at[b], acc_s.at[dst_v.at[j]], add=True)
                nj = j + NBUF

                @pl.when(nj < nch)
                def _():
                    pltpu.async_copy(
                        h_hbm.at[src_v.at[nj]], rows_v.at[b], sems[b]
                    )

            return 0

        lax.fori_loop(0, ngrp, _edge_group, 0)

        plsc.subcore_barrier()

        def _writeback(i, _):
            r = base + i * WB_CHUNK
            pltpu.sync_copy(acc_s.at[pl.ds(r, WB_CHUNK)], rows_v.at[0])
            pltpu.sync_copy(rows_v.at[0],
                            agg_hbm.at[c].at[p].at[pl.ds(r, WB_CHUNK)])
            return 0

        lax.fori_loop(0, ROWS_PER_TILE // WB_CHUNK, _writeback, 0)


def _sc_aggregate(h0, h1, src2d, dst2d):
    mesh = plsc.VectorSubcoreMesh(core_axis_name="c", subcore_axis_name="s")
    kern = pl.kernel(
        _sc_aggregate_body,
        mesh=mesh,
        out_type=jax.ShapeDtypeStruct((2, 2, ROWS, DW), jnp.float32),
        scratch_types=[
            pltpu.VMEM((NCH0, CHUNK), jnp.int32),
            pltpu.VMEM((NCH0, CHUNK), jnp.int32),
            pltpu.VMEM((NBUF, CHUNK, DW), jnp.float32),
            pltpu.VMEM_SHARED((ROWS, DW), jnp.float32),
            pltpu.SemaphoreType.DMA,
            pltpu.SemaphoreType.DMA,
            pltpu.SemaphoreType.DMA,
            pltpu.SemaphoreType.DMA,
        ],
        compiler_params=pltpu.CompilerParams(
            needs_layout_passes=False,
            use_tc_tiling_on_sc=False,
        ),
    )
    return kern(h0, h1, src2d, dst2d)


def _combine_body(hself_ref, agg_ref, out_ref):
    a_lo = agg_ref[0, 0, :, :DH] + agg_ref[1, 0, :, :DH]
    a_hi = agg_ref[0, 1, :, :DH] + agg_ref[1, 1, :, :DH]
    a = jnp.concatenate([a_lo, a_hi], axis=1)
    d = agg_ref[0, 0, :, DH:DH + 1] + agg_ref[1, 0, :, DH:DH + 1]
    out_ref[...] = hself_ref[...] + a / jnp.maximum(d, 1.0)


def _combine(h_self, agg):
    return pl.pallas_call(
        _combine_body,
        grid=(ROWS // BR,),
        in_specs=[
            pl.BlockSpec((BR, D_), lambda i: (i, 0)),
            pl.BlockSpec((2, 2, BR, DW), lambda i: (0, 0, i, 0)),
        ],
        out_specs=pl.BlockSpec((BR, D_), lambda i: (i, 0)),
        out_shape=jax.ShapeDtypeStruct((ROWS, D_), jnp.float32),
        compiler_params=pltpu.CompilerParams(
            dimension_semantics=("arbitrary",),
        ),
    )(h_self, agg)


def kernel(feat, edge_index, W_self, W_neigh):
    feat_p = jnp.pad(feat, ((0, ROWS - N_NODES_), (0, 0)))
    h_self, h0, h1 = _matmul_maxk(feat_p, W_self, W_neigh)

    src = edge_index[0]
    dst = edge_index[1]
    pad = EDGES_PAD - N_EDGES_
    src_p = jnp.concatenate([src, jnp.zeros((pad,), jnp.int32)])
    dump = DUMP_ROW + (jnp.arange(pad, dtype=jnp.int32) % (ROWS - N_NODES_))
    dst_p = jnp.concatenate([dst, dump])
    src2d = src_p.reshape(NCH_TOT, CHUNK)
    dst2d = dst_p.reshape(NCH_TOT, CHUNK)

    agg = _sc_aggregate(h0, h1, src2d, dst2d)
    out = _combine(h_self, agg)
    return out[:N_NODES_]
